# baseline (speedup 1.0000x reference)
import jax
import jax.numpy as jnp
from jax.experimental import pallas as pl

_BN = 1000


def _heads_kernel(x_ref,
                  wc_ref, bc_ref,
                  wco_ref, bco_ref,
                  wm_ref, bm_ref,
                  ws_ref, bs_ref,
                  wb_ref, bb_ref,
                  scores_ref, color_ref, material_ref, state_ref, bbox_ref):
    x = x_ref[...].astype(jnp.bfloat16)
    scores_ref[...] = jnp.dot(x, wc_ref[...], preferred_element_type=jnp.float32) + bc_ref[...]
    color_ref[...] = jnp.dot(x, wco_ref[...], preferred_element_type=jnp.float32) + bco_ref[...]
    material_ref[...] = jnp.dot(x, wm_ref[...], preferred_element_type=jnp.float32) + bm_ref[...]
    state_ref[...] = jnp.dot(x, ws_ref[...], preferred_element_type=jnp.float32) + bs_ref[...]
    bbox_ref[...] = jnp.dot(x, wb_ref[...], preferred_element_type=jnp.float32) + bb_ref[...]


def kernel(x, W_cls, b_cls, W_color, b_color, W_material, b_material,
           W_state, b_state, W_bbox, b_bbox):
    n, c = x.shape
    heads = [(W_cls, b_cls), (W_color, b_color), (W_material, b_material),
             (W_state, b_state), (W_bbox, b_bbox)]
    grid = (n // _BN,) if n % _BN == 0 else (pl.cdiv(n, _BN),)

    x_spec = pl.BlockSpec((_BN, c), lambda i: (i, 0))
    full = pl.BlockSpec(None, lambda i: (0,) * 2)

    in_specs = [x_spec]
    operands = [x]
    for W, b in heads:
        in_specs += [full, full]
        operands += [W.astype(jnp.bfloat16), b.reshape(1, -1)]

    out_shapes = tuple(jax.ShapeDtypeStruct((n, W.shape[1]), jnp.float32)
                       for W, _ in heads)
    out_specs = tuple(pl.BlockSpec((_BN, W.shape[1]), lambda i: (i, 0))
                      for W, _ in heads)

    return pl.pallas_call(
        _heads_kernel,
        grid=grid,
        in_specs=in_specs,
        out_specs=out_specs,
        out_shape=out_shapes,
    )(*operands)


# speedup vs baseline: 1.7047x; 1.7047x over previous
import jax
import jax.numpy as jnp
from jax.experimental import pallas as pl

_BN = 2048


def _heads_kernel(x_ref,
                  wc_ref, bc_ref,
                  wco_ref, bco_ref,
                  wm_ref, bm_ref,
                  ws_ref, bs_ref,
                  wb_ref, bb_ref,
                  scores_ref, color_ref, material_ref, state_ref, bbox_ref):
    x = x_ref[...].astype(jnp.bfloat16)
    dims = (((1,), (1,)), ((), ()))

    def head(w_ref, b_ref):
        y = jax.lax.dot_general(w_ref[...], x, dims,
                                preferred_element_type=jnp.float32)
        return y + b_ref[...]

    scores_ref[...] = head(wc_ref, bc_ref)
    color_ref[...] = head(wco_ref, bco_ref)
    material_ref[...] = head(wm_ref, bm_ref)
    state_ref[...] = head(ws_ref, bs_ref)
    bbox_ref[...] = head(wb_ref, bb_ref)


def kernel(x, W_cls, b_cls, W_color, b_color, W_material, b_material,
           W_state, b_state, W_bbox, b_bbox):
    n, c = x.shape
    heads = [(W_cls, b_cls), (W_color, b_color), (W_material, b_material),
             (W_state, b_state), (W_bbox, b_bbox)]
    grid = (pl.cdiv(n, _BN),)

    x_spec = pl.BlockSpec((_BN, c), lambda i: (i, 0))
    full = pl.BlockSpec(None, lambda i: (0, 0))

    in_specs = [x_spec]
    operands = [x]
    for W, b in heads:
        in_specs += [full, full]
        operands += [W.T.astype(jnp.bfloat16), b.reshape(-1, 1)]

    out_shapes = tuple(jax.ShapeDtypeStruct((W.shape[1], n), jnp.float32)
                       for W, _ in heads)
    out_specs = tuple(pl.BlockSpec((W.shape[1], _BN), lambda i: (0, i))
                      for W, _ in heads)

    outs = pl.pallas_call(
        _heads_kernel,
        grid=grid,
        in_specs=in_specs,
        out_specs=out_specs,
        out_shape=out_shapes,
    )(*operands)
    return tuple(jnp.transpose(o) for o in outs)
